# baseline (speedup 1.0000x reference)
import functools

import jax
import jax.numpy as jnp
from jax import lax
from jax.experimental import pallas as pl
from jax.experimental.pallas import tpu as pltpu
from jax.experimental.pallas import tpu_sc as plsc

N = 10000
E = 320000
F_IN = 128
HID = 128
H = 8
D = 16
OUT = 4
R = 3

BLK = 400


def _proj_body(x_ref, w_ref, b_ref, ap_ref, h_ref, a_ref):
    h = jnp.dot(x_ref[...], w_ref[...], preferred_element_type=jnp.float32)
    h = h + b_ref[...]
    h_ref[...] = h
    a_ref[...] = jnp.dot(h, ap_ref[...], preferred_element_type=jnp.float32)


def _project(x, w, b, apack):
    grid = N // BLK
    return pl.pallas_call(
        _proj_body,
        grid=(grid,),
        in_specs=[
            pl.BlockSpec((BLK, F_IN), lambda i: (i, 0)),
            pl.BlockSpec((F_IN, HID), lambda i: (0, 0)),
            pl.BlockSpec((1, HID), lambda i: (0, 0)),
            pl.BlockSpec((HID, HID), lambda i: (0, 0)),
        ],
        out_specs=[
            pl.BlockSpec((BLK, HID), lambda i: (i, 0)),
            pl.BlockSpec((BLK, HID), lambda i: (i, 0)),
        ],
        out_shape=[
            jax.ShapeDtypeStruct((N, HID), jnp.float32),
            jax.ShapeDtypeStruct((N, HID), jnp.float32),
        ],
    )(x, w, b, apack)


def _c1_body(u0_ref, u1_ref, u2_ref, d0_ref, d1_ref, d2_ref, rep_ref,
             wk_ref, bk_ref, q_ref, o0_ref, o1_ref, o2_ref, s_ref):
    @pl.when(pl.program_id(0) == 0)
    def _init():
        s_ref[...] = jnp.zeros_like(s_ref)

    row = lax.broadcasted_iota(jnp.int32, (8, 128), 0)
    col = lax.broadcasted_iota(jnp.int32, (8, 128), 1)
    acc = jnp.zeros((8, 128), jnp.float32)
    for r, (u_ref, d_ref, o_ref) in enumerate(
            ((u0_ref, d0_ref, o0_ref), (u1_ref, d1_ref, o1_ref),
             (u2_ref, d2_ref, o2_ref))):
        u = u_ref[0] + u_ref[1]
        dn = d_ref[0] + d_ref[1]
        den128 = jnp.dot(dn, rep_ref[...], preferred_element_type=jnp.float32)
        o = jnp.maximum(u, 0.0) / (den128 + 1e-16)
        o_ref[...] = o
        kt = jnp.tanh(jnp.dot(o, wk_ref[...],
                              preferred_element_type=jnp.float32) + bk_ref[...])
        spart = jnp.sum(kt * q_ref[...])
        acc = acc + jnp.where((row == 0) & (col == r), spart, 0.0)
    s_ref[...] += acc


def _c1(u_parts, d_parts, rep, wk, bk, q):
    grid = N // BLK
    u_spec = pl.BlockSpec((2, BLK, HID), lambda i: (0, i, 0))
    d_spec = pl.BlockSpec((2, BLK, 8), lambda i: (0, i, 0))
    o_spec = pl.BlockSpec((BLK, HID), lambda i: (i, 0))
    return pl.pallas_call(
        _c1_body,
        grid=(grid,),
        in_specs=[u_spec, u_spec, u_spec, d_spec, d_spec, d_spec,
                  pl.BlockSpec((8, HID), lambda i: (0, 0)),
                  pl.BlockSpec((HID, HID), lambda i: (0, 0)),
                  pl.BlockSpec((1, HID), lambda i: (0, 0)),
                  pl.BlockSpec((1, HID), lambda i: (0, 0))],
        out_specs=[o_spec, o_spec, o_spec,
                   pl.BlockSpec((8, 128), lambda i: (0, 0))],
        out_shape=[jax.ShapeDtypeStruct((N, HID), jnp.float32)] * 3
        + [jax.ShapeDtypeStruct((8, 128), jnp.float32)],
    )(*u_parts, *d_parts, rep, wk, bk, q)


def _c2_body(o0_ref, o1_ref, o2_ref, s_ref, wl_ref, bl_ref, out_ref):
    svec = s_ref[0:1, :] * (1.0 / N)
    col = lax.broadcasted_iota(jnp.int32, (1, 128), 1)
    valid = col < R
    m = jnp.max(jnp.where(valid, svec, -jnp.inf))
    e = jnp.where(valid, jnp.exp(svec - m), 0.0)
    attn = e / jnp.sum(e)
    comb = (o0_ref[...] * attn[0:1, 0:1] + o1_ref[...] * attn[0:1, 1:2]
            + o2_ref[...] * attn[0:1, 2:3])
    out_ref[...] = jnp.dot(comb, wl_ref[...],
                           preferred_element_type=jnp.float32) + bl_ref[...]


def _c2(o0, o1, o2, s, wlp, blp):
    grid = N // BLK
    o_spec = pl.BlockSpec((BLK, HID), lambda i: (i, 0))
    return pl.pallas_call(
        _c2_body,
        grid=(grid,),
        in_specs=[o_spec, o_spec, o_spec,
                  pl.BlockSpec((8, 128), lambda i: (0, 0)),
                  pl.BlockSpec((HID, HID), lambda i: (0, 0)),
                  pl.BlockSpec((1, HID), lambda i: (0, 0))],
        out_specs=o_spec,
        out_shape=jax.ShapeDtypeStruct((N, HID), jnp.float32),
    )(o0, o1, o2, s, wlp, blp)


C = 32
NBLK = E // C
NW = 32
N_PAD = 10112
STRIPE = N_PAD // 16
DEN_ROWS = 640


def _edge_body(h_ref, a48_ref,
               s0, d0, s1, d1, s2, d2,
               u0, u1, u2, dn0, dn1, dn2,
               sbufA, sbufB, dbufA, dbufB, dbuf8, asbuf, adbuf,
               hrowA, hrowB, ebuf2, evbuf, tmp,
               u_slab, den_slab, semI, semH, semA):
    core = lax.axis_index("c")
    sub = lax.axis_index("s")
    wid = sub * 2 + core
    nb = jnp.where(wid < NBLK % NW, NBLK // NW + 1, NBLK // NW)
    npairs = (NBLK // NW) // 2
    r0 = sub * STRIPE
    lane = lax.iota(jnp.int32, 16)
    tmp[pl.ds(0, 16)] = jnp.zeros((16,), jnp.float32)

    for r in range(R):
        src_ref = (s0, s1, s2)[r]
        dst_ref = (d0, d1, d2)[r]
        uout = (u0, u1, u2)[r]
        dnout = (dn0, dn1, dn2)[r]

        def _idx_issue(i, sb, db):
            eb = (wid + i * NW) * C
            pltpu.async_copy(src_ref.at[pl.ds(eb, C)], sb, semI)
            pltpu.async_copy(dst_ref.at[pl.ds(eb, C)], db, semI)

        def _idx_wait(sb, db):
            pltpu.make_async_copy(src_ref.at[pl.ds(0, C)], sb, semI).wait()
            pltpu.make_async_copy(dst_ref.at[pl.ds(0, C)], db, semI).wait()

        def _pass1(db):
            def _grp(g, c2):
                dv = db[pl.ds(g * 16, 16)]
                dbuf8[pl.ds(g * 16, 16)] = lax.shift_right_logical(dv, 4)
                offv = ((dv & 15) >> 1) * 16
                shv = (dv & 1) * 8
                for j in range(16):
                    e = g * 16 + j
                    a_s = asbuf[e, pl.ds(r * 32, 16)]
                    a_d = adbuf[e, pl.ds(r * 32 + 16, 16)]
                    al = a_s + a_d
                    al = jnp.where(al > 0, al, 0.2 * al)
                    ev = jnp.exp(al)
                    evbuf[e, pl.ds(0, 16)] = ev
                    evc = jnp.where(lane < 8, ev, 0.0)
                    tmp[pl.ds(8, 16)] = evc
                    ebuf2[e, pl.ds(offv[j], 16)] = tmp[pl.ds(8 - shv[j], 16)]
                return c2
            lax.fori_loop(0, C // 16, _grp, 0)

        def _pass2(hr):
            def _grp(g, c2):
                for j in range(16):
                    e = g * 16 + j
                    ev = evbuf[e, pl.ds(0, 16)]
                    for hh in range(H):
                        vec = hr[e, pl.ds(hh * D, D)]
                        hr[e, pl.ds(hh * D, D)] = vec * ev[hh]
                return c2
            lax.fori_loop(0, C // 16, _grp, 0)

        def _scatters(hr, db):
            pltpu.sync_copy(hr, u_slab.at[db], add=True)
            pltpu.sync_copy(ebuf2, den_slab.at[dbuf8], add=True)

            def _restore(g, c2):
                dv = db[pl.ds(g * 16, 16)]
                offv = ((dv & 15) >> 1) * 16
                for j in range(16):
                    ebuf2[g * 16 + j, pl.ds(offv[j], 16)] = (
                        jnp.zeros((16,), jnp.float32))
                return c2
            lax.fori_loop(0, C // 16, _restore, 0)

        def _half(k, sb, db, hr, pre_i, pre_sb, pre_db, cond_pre):
            _idx_wait(sb, db)
            pltpu.async_copy(h_ref.at[sb], hr, semH)
            ga1 = pltpu.async_copy(a48_ref.at[sb], asbuf, semA)
            ga2 = pltpu.async_copy(a48_ref.at[db], adbuf, semA)
            ga1.wait()
            ga2.wait()
            _pass1(db)
            if cond_pre is None:
                _idx_issue(pre_i, pre_sb, pre_db)
            else:
                @pl.when(cond_pre)
                def _():
                    _idx_issue(pre_i, pre_sb, pre_db)
            pltpu.make_async_copy(h_ref.at[pl.ds(0, C)], hr, semH).wait()
            _pass2(hr)
            _scatters(hr, db)

        def _zero_stage(i, carry):
            for cc in range(8):
                hrowA[i, pl.ds(cc * 16, 16)] = jnp.zeros((16,), jnp.float32)
                ebuf2[i, pl.ds(cc * 16, 16)] = jnp.zeros((16,), jnp.float32)
            return carry
        lax.fori_loop(0, C, _zero_stage, 0)
        for j in range(19):
            pltpu.sync_copy(hrowA, u_slab.at[pl.ds(r0 + j * C, C)])
        pltpu.sync_copy(hrowA.at[pl.ds(0, 24)],
                        u_slab.at[pl.ds(r0 + 19 * C, 24)])
        pltpu.sync_copy(ebuf2, den_slab.at[pl.ds(sub * 40, 32)])
        pltpu.sync_copy(ebuf2.at[pl.ds(0, 8)],
                        den_slab.at[pl.ds(sub * 40 + 32, 8)])
        plsc.subcore_barrier()

        _idx_issue(0, sbufA, dbufA)

        def _pair(p, carry):
            k0 = 2 * p
            _half(k0, sbufA, dbufA, hrowA, k0 + 1, sbufB, dbufB, None)
            _half(k0 + 1, sbufB, dbufB, hrowB, k0 + 2, sbufA, dbufA,
                  k0 + 2 < nb)
            return carry
        lax.fori_loop(0, npairs, _pair, 0)

        @pl.when(nb > 2 * npairs)
        def _leftover():
            _idx_wait(sbufA, dbufA)
            pltpu.async_copy(h_ref.at[sbufA], hrowA, semH)
            ga1 = pltpu.async_copy(a48_ref.at[sbufA], asbuf, semA)
            ga2 = pltpu.async_copy(a48_ref.at[dbufA], adbuf, semA)
            ga1.wait()
            ga2.wait()
            _pass1(dbufA)
            pltpu.make_async_copy(h_ref.at[pl.ds(0, C)], hrowA, semH).wait()
            _pass2(hrowA)
            _scatters(hrowA, dbufA)

        plsc.subcore_barrier()

        for j in range(19):
            pltpu.sync_copy(u_slab.at[pl.ds(r0 + j * C, C)], hrowA)
            pltpu.sync_copy(hrowA, uout.at[core, pl.ds(r0 + j * C, C)])
        pltpu.sync_copy(u_slab.at[pl.ds(r0 + 19 * C, 24)],
                        hrowA.at[pl.ds(0, 24)])
        pltpu.sync_copy(hrowA.at[pl.ds(0, 24)],
                        uout.at[core, pl.ds(r0 + 19 * C, 24)])
        pltpu.sync_copy(den_slab.at[pl.ds(sub * 40, 32)], ebuf2)
        pltpu.sync_copy(ebuf2, dnout.at[core, pl.ds(sub * 40, 32)])
        pltpu.sync_copy(den_slab.at[pl.ds(sub * 40 + 32, 8)],
                        ebuf2.at[pl.ds(0, 8)])
        pltpu.sync_copy(ebuf2.at[pl.ds(0, 8)],
                        dnout.at[core, pl.ds(sub * 40 + 32, 8)])


def _edge_phase_sc(h, a48, edges):
    mesh = plsc.VectorSubcoreMesh(core_axis_name="c", subcore_axis_name="s",
                                  num_cores=2, num_subcores=16)
    f = pl.kernel(
        _edge_body,
        out_type=[jax.ShapeDtypeStruct((2, N_PAD, HID), jnp.float32)] * 3
        + [jax.ShapeDtypeStruct((2, DEN_ROWS, 128), jnp.float32)] * 3,
        mesh=mesh,
        scratch_types=[
            pltpu.VMEM((C,), jnp.int32),
            pltpu.VMEM((C,), jnp.int32),
            pltpu.VMEM((C,), jnp.int32),
            pltpu.VMEM((C,), jnp.int32),
            pltpu.VMEM((C,), jnp.int32),
            pltpu.VMEM((C, 128), jnp.float32),
            pltpu.VMEM((C, 128), jnp.float32),
            pltpu.VMEM((C, HID), jnp.float32),
            pltpu.VMEM((C, HID), jnp.float32),
            pltpu.VMEM((C, 128), jnp.float32),
            pltpu.VMEM((C, 16), jnp.float32),
            pltpu.VMEM((32,), jnp.float32),
            pltpu.VMEM_SHARED((N_PAD, HID), jnp.float32),
            pltpu.VMEM_SHARED((DEN_ROWS, 128), jnp.float32),
            pltpu.SemaphoreType.DMA,
            pltpu.SemaphoreType.DMA,
            pltpu.SemaphoreType.DMA,
        ],
    )
    return f(h, a48,
             edges[0][0], edges[0][1], edges[1][0], edges[1][1],
             edges[2][0], edges[2][1])


def kernel(x_movie, edge_index_0, edge_index_1, edge_index_2, W_node, b_node,
           att_src_0, att_dst_0, att_src_1, att_dst_1, att_src_2, att_dst_2,
           Wk, bk, q_sem, Wl, bl):
    att_src = [att_src_0, att_src_1, att_src_2]
    att_dst = [att_dst_0, att_dst_1, att_dst_2]
    edges = [edge_index_0, edge_index_1, edge_index_2]

    hh = jnp.arange(H)
    dd = jnp.arange(D)
    rows = (hh[:, None] * D + dd[None, :]).reshape(-1)
    apack = jnp.zeros((HID, 128), jnp.float32)
    for r in range(R):
        asrc_flat = att_src[r].reshape(-1)
        adst_flat = att_dst[r].reshape(-1)
        apack = apack.at[rows, jnp.repeat(r * 32 + hh, D)].set(asrc_flat)
        apack = apack.at[rows, jnp.repeat(r * 32 + H + hh, D)].set(adst_flat)
        apack = apack.at[rows, jnp.repeat(r * 32 + 16 + hh, D)].set(adst_flat)
        apack = apack.at[rows, jnp.repeat(r * 32 + 24 + hh, D)].set(asrc_flat)

    h, a48 = _project(x_movie, W_node, b_node.reshape(1, HID), apack)

    u0, u1, u2, dn0, dn1, dn2 = _edge_phase_sc(h, a48, edges)
    u_parts = [u0, u1, u2]
    d_parts = [d.reshape(2, DEN_ROWS * 16, 8) for d in (dn0, dn1, dn2)]

    rep = jnp.zeros((8, HID), jnp.float32).at[
        jnp.repeat(hh, D), rows].set(1.0)

    o0, o1, o2, s = _c1(u_parts, d_parts, rep, Wk, bk.reshape(1, HID),
                        q_sem.reshape(1, HID))
    wlp = jnp.zeros((HID, 128), jnp.float32).at[:, :OUT].set(Wl)
    blp = jnp.zeros((1, 128), jnp.float32).at[0, :OUT].set(bl)
    out = _c2(o0, o1, o2, s, wlp, blp)
    return out[:, :OUT]


# speedup vs baseline: 1.5923x; 1.5923x over previous
import functools

import jax
import jax.numpy as jnp
from jax import lax
from jax.experimental import pallas as pl
from jax.experimental.pallas import tpu as pltpu
from jax.experimental.pallas import tpu_sc as plsc

N = 10000
E = 320000
F_IN = 128
HID = 128
H = 8
D = 16
OUT = 4
R = 3

BLK = 400


def _proj_body(x_ref, w_ref, b_ref, ap_ref, h_ref, a_ref):
    h = jnp.dot(x_ref[...], w_ref[...], preferred_element_type=jnp.float32)
    h = h + b_ref[...]
    h_ref[...] = h
    a_ref[...] = jnp.dot(h, ap_ref[...], preferred_element_type=jnp.float32)


def _project(x, w, b, apack):
    grid = N // BLK
    return pl.pallas_call(
        _proj_body,
        grid=(grid,),
        in_specs=[
            pl.BlockSpec((BLK, F_IN), lambda i: (i, 0)),
            pl.BlockSpec((F_IN, HID), lambda i: (0, 0)),
            pl.BlockSpec((1, HID), lambda i: (0, 0)),
            pl.BlockSpec((HID, HID), lambda i: (0, 0)),
        ],
        out_specs=[
            pl.BlockSpec((BLK, HID), lambda i: (i, 0)),
            pl.BlockSpec((BLK, HID), lambda i: (i, 0)),
        ],
        out_shape=[
            jax.ShapeDtypeStruct((N, HID), jnp.float32),
            jax.ShapeDtypeStruct((N, HID), jnp.float32),
        ],
    )(x, w, b, apack)


def _c1_body(u0_ref, u1_ref, u2_ref, d0_ref, d1_ref, d2_ref, rep_ref,
             wk_ref, bk_ref, q_ref, o0_ref, o1_ref, o2_ref, s_ref):
    @pl.when(pl.program_id(0) == 0)
    def _init():
        s_ref[...] = jnp.zeros_like(s_ref)

    row = lax.broadcasted_iota(jnp.int32, (8, 128), 0)
    col = lax.broadcasted_iota(jnp.int32, (8, 128), 1)
    acc = jnp.zeros((8, 128), jnp.float32)
    for r, (u_ref, d_ref, o_ref) in enumerate(
            ((u0_ref, d0_ref, o0_ref), (u1_ref, d1_ref, o1_ref),
             (u2_ref, d2_ref, o2_ref))):
        u = u_ref[0] + u_ref[1]
        dn = d_ref[0] + d_ref[1]
        den128 = jnp.dot(dn, rep_ref[...], preferred_element_type=jnp.float32)
        o = jnp.maximum(u, 0.0) / (den128 + 1e-16)
        o_ref[...] = o
        kt = jnp.tanh(jnp.dot(o, wk_ref[...],
                              preferred_element_type=jnp.float32) + bk_ref[...])
        spart = jnp.sum(kt * q_ref[...])
        acc = acc + jnp.where((row == 0) & (col == r), spart, 0.0)
    s_ref[...] += acc


def _c1(u_parts, d_parts, rep, wk, bk, q):
    grid = N // BLK
    u_spec = pl.BlockSpec((2, BLK, HID), lambda i: (0, i, 0))
    d_spec = pl.BlockSpec((2, BLK, 16), lambda i: (0, i, 0))
    o_spec = pl.BlockSpec((BLK, HID), lambda i: (i, 0))
    return pl.pallas_call(
        _c1_body,
        grid=(grid,),
        in_specs=[u_spec, u_spec, u_spec, d_spec, d_spec, d_spec,
                  pl.BlockSpec((16, HID), lambda i: (0, 0)),
                  pl.BlockSpec((HID, HID), lambda i: (0, 0)),
                  pl.BlockSpec((1, HID), lambda i: (0, 0)),
                  pl.BlockSpec((1, HID), lambda i: (0, 0))],
        out_specs=[o_spec, o_spec, o_spec,
                   pl.BlockSpec((8, 128), lambda i: (0, 0))],
        out_shape=[jax.ShapeDtypeStruct((N, HID), jnp.float32)] * 3
        + [jax.ShapeDtypeStruct((8, 128), jnp.float32)],
    )(*u_parts, *d_parts, rep, wk, bk, q)


def _c2_body(o0_ref, o1_ref, o2_ref, s_ref, wl_ref, bl_ref, out_ref):
    svec = s_ref[0:1, :] * (1.0 / N)
    col = lax.broadcasted_iota(jnp.int32, (1, 128), 1)
    valid = col < R
    m = jnp.max(jnp.where(valid, svec, -jnp.inf))
    e = jnp.where(valid, jnp.exp(svec - m), 0.0)
    attn = e / jnp.sum(e)
    comb = (o0_ref[...] * attn[0:1, 0:1] + o1_ref[...] * attn[0:1, 1:2]
            + o2_ref[...] * attn[0:1, 2:3])
    out_ref[...] = jnp.dot(comb, wl_ref[...],
                           preferred_element_type=jnp.float32) + bl_ref[...]


def _c2(o0, o1, o2, s, wlp, blp):
    grid = N // BLK
    o_spec = pl.BlockSpec((BLK, HID), lambda i: (i, 0))
    return pl.pallas_call(
        _c2_body,
        grid=(grid,),
        in_specs=[o_spec, o_spec, o_spec,
                  pl.BlockSpec((8, 128), lambda i: (0, 0)),
                  pl.BlockSpec((HID, HID), lambda i: (0, 0)),
                  pl.BlockSpec((1, HID), lambda i: (0, 0))],
        out_specs=o_spec,
        out_shape=jax.ShapeDtypeStruct((N, HID), jnp.float32),
    )(o0, o1, o2, s, wlp, blp)


C = 64
NBLK = E // C
NW = 32
N_PAD = 10240
STRIPE = N_PAD // 16


def _edge_body(h_ref, a48_ref,
               e0, e1, e2,
               u0, u1, u2, dn0, dn1, dn2,
               ibuf, dbufW, dbuf8, asbuf, adbuf, hrow, ebuf2,
               u_slab, den_slab, sem):
    core = lax.axis_index("c")
    sub = lax.axis_index("s")
    wid = sub * 2 + core
    nb = jnp.where(wid < NBLK % NW, NBLK // NW + 1, NBLK // NW)
    r0 = sub * STRIPE

    for r in range(R):
        ep_ref = (e0, e1, e2)[r]

        uout = (u0, u1, u2)[r]
        dnout = (dn0, dn1, dn2)[r]

        def _zero_stage(i, carry):
            for cc in range(8):
                hrow[i, pl.ds(cc * 16, 16)] = jnp.zeros((16,), jnp.float32)
                ebuf2[i, pl.ds(cc * 16, 16)] = jnp.zeros((16,), jnp.float32)
            return carry
        lax.fori_loop(0, C, _zero_stage, 0)
        for j in range(10):
            pltpu.sync_copy(hrow, u_slab.at[pl.ds(r0 + j * C, C)])
        pltpu.sync_copy(ebuf2, den_slab.at[pl.ds(sub * 80, 64)])
        pltpu.sync_copy(ebuf2.at[pl.ds(0, 16)],
                        den_slab.at[pl.ds(sub * 80 + 64, 16)])
        plsc.subcore_barrier()

        def _chunk(i, carry):
            cid = wid + i * NW
            pltpu.async_copy(ep_ref.at[cid], ibuf, sem).wait()
            g1 = pltpu.async_copy(a48_ref.at[ibuf.at[0, pl.ds(0, C)]],
                                  asbuf, sem)
            g2 = pltpu.async_copy(a48_ref.at[ibuf.at[0, pl.ds(C, C)]],
                                  adbuf, sem)
            g3 = pltpu.async_copy(h_ref.at[ibuf.at[0, pl.ds(0, C)]],
                                  hrow, sem)
            g1.wait()
            g2.wait()
            g3.wait()

            def _group(g, carry2):
                dv = ibuf[0, pl.ds(C + g * 16, 16)]
                dbufW[pl.ds(g * 16, 16)] = dv
                dbuf8[pl.ds(g * 16, 16)] = lax.shift_right_logical(dv, 3)
                offv = (dv & 7) * 16
                for j in range(16):
                    e = g * 16 + j
                    a_s = asbuf[e, pl.ds(r * 32, 16)]
                    a_d = adbuf[e, pl.ds(r * 32 + 16, 16)]
                    al = a_s + a_d
                    al = jnp.where(al > 0, al, 0.2 * al)
                    ev = jnp.exp(al)
                    ebuf2[e, pl.ds(offv[j], 16)] = ev
                    for hh in range(H):
                        cval = ev[hh]
                        vec = hrow[e, pl.ds(hh * D, D)]
                        hrow[e, pl.ds(hh * D, D)] = vec * cval
                return carry2
            lax.fori_loop(0, C // 16, _group, 0)
            s1 = pltpu.async_copy(hrow, u_slab.at[dbufW], sem, add=True)
            s2 = pltpu.async_copy(ebuf2, den_slab.at[dbuf8], sem, add=True)
            s1.wait()
            s2.wait()

            def _restore(g, carry2):
                dv = dbufW[pl.ds(g * 16, 16)]
                offv = (dv & 7) * 16
                for j in range(16):
                    ebuf2[g * 16 + j, pl.ds(offv[j], 16)] = (
                        jnp.zeros((16,), jnp.float32))
                return carry2
            lax.fori_loop(0, C // 16, _restore, 0)
            return carry
        lax.fori_loop(0, nb, _chunk, 0)
        plsc.subcore_barrier()

        for j in range(10):
            pltpu.sync_copy(u_slab.at[pl.ds(r0 + j * C, C)], hrow)
            pltpu.sync_copy(hrow, uout.at[core, pl.ds(r0 + j * C, C)])
        pltpu.sync_copy(den_slab.at[pl.ds(sub * 80, 64)], ebuf2)
        pltpu.sync_copy(ebuf2, dnout.at[core, pl.ds(sub * 80, 64)])
        pltpu.sync_copy(den_slab.at[pl.ds(sub * 80 + 64, 16)],
                        ebuf2.at[pl.ds(0, 16)])
        pltpu.sync_copy(ebuf2.at[pl.ds(0, 16)],
                        dnout.at[core, pl.ds(sub * 80 + 64, 16)])


def _edge_phase_sc(h, a48, edges):
    mesh = plsc.VectorSubcoreMesh(core_axis_name="c", subcore_axis_name="s",
                                  num_cores=2, num_subcores=16)
    f = pl.kernel(
        _edge_body,
        out_type=[jax.ShapeDtypeStruct((2, N_PAD, HID), jnp.float32)] * 3
        + [jax.ShapeDtypeStruct((2, N_PAD // 8, 128), jnp.float32)] * 3,
        mesh=mesh,
        scratch_types=[
            pltpu.VMEM((1, 128), jnp.int32),
            pltpu.VMEM((C,), jnp.int32),
            pltpu.VMEM((C,), jnp.int32),
            pltpu.VMEM((C, 128), jnp.float32),
            pltpu.VMEM((C, 128), jnp.float32),
            pltpu.VMEM((C, HID), jnp.float32),
            pltpu.VMEM((C, 128), jnp.float32),
            pltpu.VMEM_SHARED((N_PAD, HID), jnp.float32),
            pltpu.VMEM_SHARED((N_PAD // 8, 128), jnp.float32),
            pltpu.SemaphoreType.DMA,
        ],
    )
    epacks = []
    for ed in edges:
        s2 = ed[0].reshape(NBLK, C)
        d2 = ed[1].reshape(NBLK, C)
        epacks.append(jnp.concatenate([s2, d2], 1).reshape(NBLK, 1, 2 * C))
    return f(h, a48, *epacks)


def kernel(x_movie, edge_index_0, edge_index_1, edge_index_2, W_node, b_node,
           att_src_0, att_dst_0, att_src_1, att_dst_1, att_src_2, att_dst_2,
           Wk, bk, q_sem, Wl, bl):
    att_src = [att_src_0, att_src_1, att_src_2]
    att_dst = [att_dst_0, att_dst_1, att_dst_2]
    edges = [edge_index_0, edge_index_1, edge_index_2]

    hh = jnp.arange(H)
    dd = jnp.arange(D)
    rows = (hh[:, None] * D + dd[None, :]).reshape(-1)
    apack = jnp.zeros((HID, 128), jnp.float32)
    for r in range(R):
        asrc_flat = att_src[r].reshape(-1)
        adst_flat = att_dst[r].reshape(-1)
        apack = apack.at[rows, jnp.repeat(r * 32 + hh, D)].set(asrc_flat)
        apack = apack.at[rows, jnp.repeat(r * 32 + H + hh, D)].set(adst_flat)
        apack = apack.at[rows, jnp.repeat(r * 32 + 16 + hh, D)].set(adst_flat)
        apack = apack.at[rows, jnp.repeat(r * 32 + 24 + hh, D)].set(asrc_flat)

    h, a48 = _project(x_movie, W_node, b_node.reshape(1, HID), apack)

    u0, u1, u2, dn0, dn1, dn2 = _edge_phase_sc(h, a48, edges)
    u_parts = [u0, u1, u2]
    d_parts = [d.reshape(2, N_PAD, 16) for d in (dn0, dn1, dn2)]

    rep = jnp.zeros((16, HID), jnp.float32).at[
        jnp.repeat(hh, D), rows].set(1.0)

    o0, o1, o2, s = _c1(u_parts, d_parts, rep, Wk, bk.reshape(1, HID),
                        q_sem.reshape(1, HID))
    wlp = jnp.zeros((HID, 128), jnp.float32).at[:, :OUT].set(Wl)
    blp = jnp.zeros((1, 128), jnp.float32).at[0, :OUT].set(bl)
    out = _c2(o0, o1, o2, s, wlp, blp)
    return out[:, :OUT]


# speedup vs baseline: 1.6110x; 1.0117x over previous
import functools

import jax
import jax.numpy as jnp
from jax import lax
from jax.experimental import pallas as pl
from jax.experimental.pallas import tpu as pltpu
from jax.experimental.pallas import tpu_sc as plsc

N = 10000
E = 320000
F_IN = 128
HID = 128
H = 8
D = 16
OUT = 4
R = 3

BLK = 400


def _proj_body(x_ref, w_ref, b_ref, ap_ref, h_ref, a_ref):
    h = jnp.dot(x_ref[...], w_ref[...], preferred_element_type=jnp.float32)
    h = h + b_ref[...]
    h_ref[...] = h
    a_ref[...] = jnp.dot(h, ap_ref[...], preferred_element_type=jnp.float32)


def _project(x, w, b, apack):
    grid = N // BLK
    return pl.pallas_call(
        _proj_body,
        grid=(grid,),
        in_specs=[
            pl.BlockSpec((BLK, F_IN), lambda i: (i, 0)),
            pl.BlockSpec((F_IN, HID), lambda i: (0, 0)),
            pl.BlockSpec((1, HID), lambda i: (0, 0)),
            pl.BlockSpec((HID, HID), lambda i: (0, 0)),
        ],
        out_specs=[
            pl.BlockSpec((BLK, HID), lambda i: (i, 0)),
            pl.BlockSpec((BLK, HID), lambda i: (i, 0)),
        ],
        out_shape=[
            jax.ShapeDtypeStruct((N, HID), jnp.float32),
            jax.ShapeDtypeStruct((N, HID), jnp.float32),
        ],
    )(x, w, b, apack)


def _c1_body(u0_ref, u1_ref, u2_ref, d0_ref, d1_ref, d2_ref, rep_ref,
             wk_ref, bk_ref, q_ref, o0_ref, o1_ref, o2_ref, s_ref):
    @pl.when(pl.program_id(0) == 0)
    def _init():
        s_ref[...] = jnp.zeros_like(s_ref)

    row = lax.broadcasted_iota(jnp.int32, (8, 128), 0)
    col = lax.broadcasted_iota(jnp.int32, (8, 128), 1)
    acc = jnp.zeros((8, 128), jnp.float32)
    for r, (u_ref, d_ref, o_ref) in enumerate(
            ((u0_ref, d0_ref, o0_ref), (u1_ref, d1_ref, o1_ref),
             (u2_ref, d2_ref, o2_ref))):
        u = u_ref[0] + u_ref[1]
        dn = d_ref[0] + d_ref[1]
        den128 = jnp.dot(dn, rep_ref[...], preferred_element_type=jnp.float32)
        o = jnp.maximum(u, 0.0) / (den128 + 1e-16)
        o_ref[...] = o
        kt = jnp.tanh(jnp.dot(o, wk_ref[...],
                              preferred_element_type=jnp.float32) + bk_ref[...])
        spart = jnp.sum(kt * q_ref[...])
        acc = acc + jnp.where((row == 0) & (col == r), spart, 0.0)
    s_ref[...] += acc


def _c1(u_parts, d_parts, rep, wk, bk, q):
    grid = N // BLK
    u_spec = pl.BlockSpec((2, BLK, HID), lambda i: (0, i, 0))
    d_spec = pl.BlockSpec((2, BLK, 16), lambda i: (0, i, 0))
    o_spec = pl.BlockSpec((BLK, HID), lambda i: (i, 0))
    return pl.pallas_call(
        _c1_body,
        grid=(grid,),
        in_specs=[u_spec, u_spec, u_spec, d_spec, d_spec, d_spec,
                  pl.BlockSpec((16, HID), lambda i: (0, 0)),
                  pl.BlockSpec((HID, HID), lambda i: (0, 0)),
                  pl.BlockSpec((1, HID), lambda i: (0, 0)),
                  pl.BlockSpec((1, HID), lambda i: (0, 0))],
        out_specs=[o_spec, o_spec, o_spec,
                   pl.BlockSpec((8, 128), lambda i: (0, 0))],
        out_shape=[jax.ShapeDtypeStruct((N, HID), jnp.float32)] * 3
        + [jax.ShapeDtypeStruct((8, 128), jnp.float32)],
    )(*u_parts, *d_parts, rep, wk, bk, q)


def _c2_body(o0_ref, o1_ref, o2_ref, s_ref, wl_ref, bl_ref, out_ref):
    svec = s_ref[0:1, :] * (1.0 / N)
    col = lax.broadcasted_iota(jnp.int32, (1, 128), 1)
    valid = col < R
    m = jnp.max(jnp.where(valid, svec, -jnp.inf))
    e = jnp.where(valid, jnp.exp(svec - m), 0.0)
    attn = e / jnp.sum(e)
    comb = (o0_ref[...] * attn[0:1, 0:1] + o1_ref[...] * attn[0:1, 1:2]
            + o2_ref[...] * attn[0:1, 2:3])
    out_ref[...] = jnp.dot(comb, wl_ref[...],
                           preferred_element_type=jnp.float32) + bl_ref[...]


def _c2(o0, o1, o2, s, wlp, blp):
    grid = N // BLK
    o_spec = pl.BlockSpec((BLK, HID), lambda i: (i, 0))
    return pl.pallas_call(
        _c2_body,
        grid=(grid,),
        in_specs=[o_spec, o_spec, o_spec,
                  pl.BlockSpec((8, 128), lambda i: (0, 0)),
                  pl.BlockSpec((HID, HID), lambda i: (0, 0)),
                  pl.BlockSpec((1, HID), lambda i: (0, 0))],
        out_specs=o_spec,
        out_shape=jax.ShapeDtypeStruct((N, HID), jnp.float32),
    )(o0, o1, o2, s, wlp, blp)


C = 64
NBLK = E // C
NW = 32
N_PAD = 10240
STRIPE = N_PAD // 16


def _edge_body(h_ref, a48_ref,
               s0, d0, s1, d1, s2, d2,
               u0, u1, u2, dn0, dn1, dn2,
               sbuf, dbuf, dbuf8, asbuf, adbuf, hrow, ebuf2,
               u_slab, den_slab, sem):
    core = lax.axis_index("c")
    sub = lax.axis_index("s")
    wid = sub * 2 + core
    nb = jnp.where(wid < NBLK % NW, NBLK // NW + 1, NBLK // NW)
    r0 = sub * STRIPE

    for r in range(R):
        src_ref = (s0, s1, s2)[r]
        dst_ref = (d0, d1, d2)[r]

        uout = (u0, u1, u2)[r]
        dnout = (dn0, dn1, dn2)[r]

        def _zero_stage(i, carry):
            for cc in range(8):
                hrow[i, pl.ds(cc * 16, 16)] = jnp.zeros((16,), jnp.float32)
                ebuf2[i, pl.ds(cc * 16, 16)] = jnp.zeros((16,), jnp.float32)
            return carry
        lax.fori_loop(0, C, _zero_stage, 0)
        for j in range(10):
            pltpu.sync_copy(hrow, u_slab.at[pl.ds(r0 + j * C, C)])
        pltpu.sync_copy(ebuf2, den_slab.at[pl.ds(sub * 80, 64)])
        pltpu.sync_copy(ebuf2.at[pl.ds(0, 16)],
                        den_slab.at[pl.ds(sub * 80 + 64, 16)])
        plsc.subcore_barrier()

        def _chunk(i, carry):
            eb = (wid + i * NW) * C
            i1 = pltpu.async_copy(src_ref.at[pl.ds(eb, C)], sbuf, sem)
            i2 = pltpu.async_copy(dst_ref.at[pl.ds(eb, C)], dbuf, sem)
            i1.wait()
            i2.wait()
            g1 = pltpu.async_copy(a48_ref.at[sbuf], asbuf, sem)
            g2 = pltpu.async_copy(a48_ref.at[dbuf], adbuf, sem)
            g3 = pltpu.async_copy(h_ref.at[sbuf], hrow, sem)
            g1.wait()
            g2.wait()
            g3.wait()

            def _group(g, carry2):
                dv = dbuf[pl.ds(g * 16, 16)]
                dbuf8[pl.ds(g * 16, 16)] = lax.shift_right_logical(dv, 3)
                offv = (dv & 7) * 16
                for j in range(16):
                    e = g * 16 + j
                    a_s = asbuf[e, pl.ds(r * 32, 16)]
                    a_d = adbuf[e, pl.ds(r * 32 + 16, 16)]
                    al = a_s + a_d
                    al = jnp.where(al > 0, al, 0.2 * al)
                    ev = jnp.exp(al)
                    ebuf2[e, pl.ds(offv[j], 16)] = ev
                    for hh in range(H):
                        cval = ev[hh]
                        vec = hrow[e, pl.ds(hh * D, D)]
                        hrow[e, pl.ds(hh * D, D)] = vec * cval
                return carry2
            lax.fori_loop(0, C // 16, _group, 0)
            pltpu.sync_copy(hrow, u_slab.at[dbuf], add=True)
            pltpu.sync_copy(ebuf2, den_slab.at[dbuf8], add=True)

            def _restore(g, carry2):
                dv = dbuf[pl.ds(g * 16, 16)]
                offv = (dv & 7) * 16
                for j in range(16):
                    ebuf2[g * 16 + j, pl.ds(offv[j], 16)] = (
                        jnp.zeros((16,), jnp.float32))
                return carry2
            lax.fori_loop(0, C // 16, _restore, 0)
            return carry
        lax.fori_loop(0, nb, _chunk, 0)
        plsc.subcore_barrier()

        for j in range(10):
            pltpu.sync_copy(u_slab.at[pl.ds(r0 + j * C, C)], hrow)
            pltpu.sync_copy(hrow, uout.at[core, pl.ds(r0 + j * C, C)])
        pltpu.sync_copy(den_slab.at[pl.ds(sub * 80, 64)], ebuf2)
        pltpu.sync_copy(ebuf2, dnout.at[core, pl.ds(sub * 80, 64)])
        pltpu.sync_copy(den_slab.at[pl.ds(sub * 80 + 64, 16)],
                        ebuf2.at[pl.ds(0, 16)])
        pltpu.sync_copy(ebuf2.at[pl.ds(0, 16)],
                        dnout.at[core, pl.ds(sub * 80 + 64, 16)])


def _edge_phase_sc(h, a48, edges):
    mesh = plsc.VectorSubcoreMesh(core_axis_name="c", subcore_axis_name="s",
                                  num_cores=2, num_subcores=16)
    f = pl.kernel(
        _edge_body,
        out_type=[jax.ShapeDtypeStruct((2, N_PAD, HID), jnp.float32)] * 3
        + [jax.ShapeDtypeStruct((2, N_PAD // 8, 128), jnp.float32)] * 3,
        mesh=mesh,
        scratch_types=[
            pltpu.VMEM((C,), jnp.int32),
            pltpu.VMEM((C,), jnp.int32),
            pltpu.VMEM((C,), jnp.int32),
            pltpu.VMEM((C, 128), jnp.float32),
            pltpu.VMEM((C, 128), jnp.float32),
            pltpu.VMEM((C, HID), jnp.float32),
            pltpu.VMEM((C, 128), jnp.float32),
            pltpu.VMEM_SHARED((N_PAD, HID), jnp.float32),
            pltpu.VMEM_SHARED((N_PAD // 8, 128), jnp.float32),
            pltpu.SemaphoreType.DMA,
        ],
    )
    return f(h, a48,
             edges[0][0], edges[0][1], edges[1][0], edges[1][1],
             edges[2][0], edges[2][1])


def kernel(x_movie, edge_index_0, edge_index_1, edge_index_2, W_node, b_node,
           att_src_0, att_dst_0, att_src_1, att_dst_1, att_src_2, att_dst_2,
           Wk, bk, q_sem, Wl, bl):
    att_src = [att_src_0, att_src_1, att_src_2]
    att_dst = [att_dst_0, att_dst_1, att_dst_2]
    edges = [edge_index_0, edge_index_1, edge_index_2]

    hh = jnp.arange(H)
    dd = jnp.arange(D)
    rows = (hh[:, None] * D + dd[None, :]).reshape(-1)
    apack = jnp.zeros((HID, 128), jnp.float32)
    for r in range(R):
        asrc_flat = att_src[r].reshape(-1)
        adst_flat = att_dst[r].reshape(-1)
        apack = apack.at[rows, jnp.repeat(r * 32 + hh, D)].set(asrc_flat)
        apack = apack.at[rows, jnp.repeat(r * 32 + H + hh, D)].set(adst_flat)
        apack = apack.at[rows, jnp.repeat(r * 32 + 16 + hh, D)].set(adst_flat)
        apack = apack.at[rows, jnp.repeat(r * 32 + 24 + hh, D)].set(asrc_flat)

    h, a48 = _project(x_movie, W_node, b_node.reshape(1, HID), apack)

    u0, u1, u2, dn0, dn1, dn2 = _edge_phase_sc(h, a48, edges)
    u_parts = [u0, u1, u2]
    d_parts = [d.reshape(2, N_PAD, 16) for d in (dn0, dn1, dn2)]

    rep = jnp.zeros((16, HID), jnp.float32).at[
        jnp.repeat(hh, D), rows].set(1.0)

    o0, o1, o2, s = _c1(u_parts, d_parts, rep, Wk, bk.reshape(1, HID),
                        q_sem.reshape(1, HID))
    wlp = jnp.zeros((HID, 128), jnp.float32).at[:, :OUT].set(Wl)
    blp = jnp.zeros((1, 128), jnp.float32).at[0, :OUT].set(bl)
    out = _c2(o0, o1, o2, s, wlp, blp)
    return out[:, :OUT]
